# baseline (speedup 1.0000x reference)
import functools

import jax
import jax.numpy as jnp
from jax import lax
from jax.experimental import pallas as pl
from jax.experimental.pallas import tpu as pltpu
from jax.experimental.pallas import tpu_sc as plsc

F32 = jnp.float32
BF16 = jnp.bfloat16
N_NODES = 10000
N_EDGES = 320000
LAT = 32
PACK = 4
NPK = N_NODES // PACK

NW = 32
E_PAD = 327680
EW = E_PAD // NW
EPP = E_PAD // PACK
EDGE_BLK = 2048
E8 = N_EDGES // 8
E8_BLK = 2000
TAB_ROWS = N_NODES + 16

G_SUP = 640
G_NSUP = EW // G_SUP
S_SUP = 1280
S_NSUP = EW // S_SUP
CH = 128


@functools.cache
def _mesh():
    return plsc.VectorSubcoreMesh(
        core_axis_name="c", subcore_axis_name="s", num_cores=2, num_subcores=16)


def _bd(w, n):
    return jax.scipy.linalg.block_diag(*([w] * n))


def _tile(v, n):
    return jnp.tile(v, n)[None, :]


def _leaky(u):
    return jnp.maximum(u, 0.01 * u)


def _ln_groups(v, m, g, b):
    mu = jnp.dot(v, m, preferred_element_type=F32)
    d = v - mu
    var = jnp.dot(d * d, m, preferred_element_type=F32)
    return d * lax.rsqrt(var + 1e-5) * g + b


def _mlp2_body(x_ref, a1_ref, b1_ref, a2_ref, b2_ref, m_ref, g_ref, be_ref, o_ref):
    u = jnp.dot(x_ref[...], a1_ref[...], preferred_element_type=F32) + b1_ref[...]
    t = _leaky(u)
    v = jnp.dot(t, a2_ref[...], preferred_element_type=F32) + b2_ref[...]
    o_ref[...] = _ln_groups(v, m_ref[...], g_ref[...], be_ref[...])


def _node_encode(x4, a1, b1, a2, b2, m, g, be):
    return pl.pallas_call(
        _mlp2_body,
        out_shape=jax.ShapeDtypeStruct((NPK, 128), F32),
    )(x4, a1, b1, a2, b2, m, g, be)


def _edge_encode(ea8, a1, b1, a2, b2, m, g, be):
    grid = E8 // E8_BLK
    full = lambda shape: pl.BlockSpec(shape, lambda i: (0, 0))
    return pl.pallas_call(
        _mlp2_body,
        grid=(grid,),
        in_specs=[
            pl.BlockSpec((E8_BLK, 128), lambda i: (i, 0)),
            full((128, 256)), full((1, 256)), full((256, 256)), full((1, 256)),
            full((256, 256)), full((1, 256)), full((1, 256)),
        ],
        out_specs=pl.BlockSpec((E8_BLK, 256), lambda i: (i, 0)),
        out_shape=jax.ShapeDtypeStruct((EPP // 2, 256), F32),
    )(ea8, a1, b1, a2, b2, m, g, be)


def _edge_mlp_body(he_ref, gs_ref, gd_ref, aa_ref, ab_ref, ac_ref, b1_ref,
                   ao_ref, b2_ref, m_ref, g_ref, be_ref, o_ref):
    he = he_ref[...]
    u = (jnp.dot(he, aa_ref[...], preferred_element_type=F32)
         + jnp.dot(gs_ref[...].astype(F32), ab_ref[...], preferred_element_type=F32)
         + jnp.dot(gd_ref[...].astype(F32), ac_ref[...], preferred_element_type=F32)
         + b1_ref[...])
    t = _leaky(u)
    v = jnp.dot(t, ao_ref[...], preferred_element_type=F32) + b2_ref[...]
    o_ref[...] = _ln_groups(v, m_ref[...], g_ref[...], be_ref[...]) + he


def _edge_mlp(he_p, gs_p, gd_p, aa, ab, ac, b1, ao, b2, m, g, be):
    grid = EPP // EDGE_BLK
    data = pl.BlockSpec((EDGE_BLK, 128), lambda i: (i, 0))
    full = lambda shape: pl.BlockSpec(shape, lambda i: (0, 0))
    return pl.pallas_call(
        _edge_mlp_body,
        grid=(grid,),
        in_specs=[data, data, data,
                  full((128, 128)), full((128, 128)), full((128, 128)),
                  full((1, 128)), full((128, 128)), full((1, 128)),
                  full((128, 128)), full((1, 128)), full((1, 128))],
        out_specs=data,
        out_shape=jax.ShapeDtypeStruct((EPP, 128), F32),
    )(he_p, gs_p, gd_p, aa, ab, ac, b1, ao, b2, m, g, be)


def _node_upd_body(hn_ref, p0_ref, p1_ref, a1_ref, a2_ref, b1_ref, ao_ref,
                   b2_ref, m_ref, g_ref, be_ref, o_ref):
    hn = hn_ref[...]
    pe = p0_ref[...] + p1_ref[...]
    u = (jnp.dot(hn, a1_ref[...], preferred_element_type=F32)
         + jnp.dot(pe, a2_ref[...], preferred_element_type=F32)
         + b1_ref[...])
    t = _leaky(u)
    v = jnp.dot(t, ao_ref[...], preferred_element_type=F32) + b2_ref[...]
    o_ref[...] = _ln_groups(v, m_ref[...], g_ref[...], be_ref[...]) + hn


def _node_update(hn_p, p0_p, p1_p, a1, a2, b1, ao, b2, m, g, be):
    return pl.pallas_call(
        _node_upd_body,
        out_shape=jax.ShapeDtypeStruct((NPK, 128), F32),
    )(hn_p, p0_p, p1_p, a1, a2, b1, ao, b2, m, g, be)


def _dec_body(hn_ref, a1_ref, b1_ref, ao_ref, b2_ref, o_ref):
    u = jnp.dot(hn_ref[...], a1_ref[...], preferred_element_type=F32) + b1_ref[...]
    t = _leaky(u)
    o_ref[...] = jnp.dot(t, ao_ref[...], preferred_element_type=F32) + b2_ref[...]


def _decode(hn_p, a1, b1, ao, b2):
    return pl.pallas_call(
        _dec_body,
        out_shape=jax.ShapeDtypeStruct((NPK, 4 * PACK), F32),
    )(hn_p, a1, b1, ao, b2)


def _sc_gather(tab, src_g, dst_g):
    @functools.partial(
        pl.kernel,
        out_type=(jax.ShapeDtypeStruct((E_PAD, LAT), BF16),
                  jax.ShapeDtypeStruct((E_PAD, LAT), BF16)),
        mesh=_mesh(),
        scratch_types=[
            pltpu.VMEM((EW,), jnp.int32),
            pltpu.VMEM((EW,), jnp.int32),
            pltpu.VMEM((G_SUP, LAT), BF16),
            pltpu.VMEM((G_SUP, LAT), BF16),
            pltpu.VMEM((G_SUP, LAT), BF16),
            pltpu.VMEM((G_SUP, LAT), BF16),
            pltpu.VMEM_SHARED((N_NODES, LAT), BF16),
            pltpu.SemaphoreType.DMA,
            pltpu.SemaphoreType.DMA,
            pltpu.SemaphoreType.DMA,
        ],
        compiler_params=pltpu.CompilerParams(use_tc_tiling_on_sc=False),
    )
    def k(tab_h, src_h, dst_h, gs_h, gd_h, idxs, idxd,
          bs0, bs1, bd0, bd1, tab_sp, semS, semD, semW):
        cid = lax.axis_index("c")
        sid = lax.axis_index("s")
        wbase = (sid * 2 + cid) * EW

        @pl.when(sid == 0)
        def _():
            pltpu.sync_copy(tab_h, tab_sp)

        pltpu.sync_copy(src_h.at[pl.ds(wbase, EW)], idxs)
        pltpu.sync_copy(dst_h.at[pl.ds(wbase, EW)], idxd)
        plsc.subcore_barrier()
        bufs = (bs0, bs1)
        bufd = (bd0, bd1)

        def issue_gathers(s):
            ds_ = []
            for j in range(G_SUP // CH):
                o = s * G_SUP + j * CH
                ds_.append(pltpu.async_copy(
                    tab_sp.at[idxs.at[pl.ds(o, CH)]],
                    bufs[s % 2].at[pl.ds(j * CH, CH)], semS))
                ds_.append(pltpu.async_copy(
                    tab_sp.at[idxd.at[pl.ds(o, CH)]],
                    bufd[s % 2].at[pl.ds(j * CH, CH)], semD))
            return ds_

        def issue_writes(s):
            o = wbase + s * G_SUP
            return [pltpu.async_copy(bufs[s % 2], gs_h.at[pl.ds(o, G_SUP)], semW),
                    pltpu.async_copy(bufd[s % 2], gd_h.at[pl.ds(o, G_SUP)], semW)]

        g = {0: issue_gathers(0)}
        w = {}
        for s in range(G_NSUP):
            if s + 1 < G_NSUP:
                if s >= 1:
                    for d in w[s - 1]:
                        d.wait()
                g[s + 1] = issue_gathers(s + 1)
            for d in g[s]:
                d.wait()
            w[s] = issue_writes(s)
        for d in w[G_NSUP - 2] + w[G_NSUP - 1]:
            d.wait()

    return k(tab, src_g, dst_g)


def _sc_scatter(he_flat, dst_s2, zeros):
    @functools.partial(
        pl.kernel,
        out_type=(jax.ShapeDtypeStruct((N_NODES, LAT), F32),
                  jax.ShapeDtypeStruct((N_NODES, LAT), F32)),
        mesh=_mesh(),
        scratch_types=[
            pltpu.VMEM((EW // CH, CH), jnp.int32),
            pltpu.VMEM((S_SUP, LAT), F32),
            pltpu.VMEM((S_SUP, LAT), F32),
            pltpu.VMEM_SHARED((TAB_ROWS, LAT), F32),
            pltpu.SemaphoreType.DMA,
            pltpu.SemaphoreType.DMA,
        ],
        compiler_params=pltpu.CompilerParams(use_tc_tiling_on_sc=False),
    )
    def k(he_h, dst2_h, z_h, p0_h, p1_h, idx2, r0, r1, table, semL, semA):
        cid = lax.axis_index("c")
        sid = lax.axis_index("s")
        wid = sid * 2 + cid

        @pl.when(sid == 0)
        def _():
            pltpu.sync_copy(z_h, table)

        plsc.subcore_barrier()
        pltpu.sync_copy(dst2_h.at[pl.ds(wid * (EW // CH), EW // CH)], idx2)
        rows = (r0, r1)

        def issue_load(s):
            return pltpu.async_copy(
                he_h.at[pl.ds(wid * EW + s * S_SUP, S_SUP)], rows[s % 2], semL)

        def issue_adds(s):
            ds_ = []
            for j in range(S_SUP // CH):
                ds_.append(pltpu.async_copy(
                    rows[s % 2].at[pl.ds(j * CH, CH)],
                    table.at[idx2.at[s * (S_SUP // CH) + j]], semA, add=True))
            return ds_

        ld = {0: issue_load(0)}
        ad = {}
        for s in range(S_NSUP):
            ld[s].wait()
            if s + 1 < S_NSUP:
                if s >= 1:
                    for d in ad[s - 1]:
                        d.wait()
                ld[s + 1] = issue_load(s + 1)
            ad[s] = issue_adds(s)
        for d in ad[S_NSUP - 2] + ad[S_NSUP - 1]:
            d.wait()
        plsc.subcore_barrier()

        rws = N_NODES // 16
        r0o = sid * rws

        @pl.when(cid == 0)
        def _():
            pltpu.sync_copy(table.at[pl.ds(r0o, rws)], p0_h.at[pl.ds(r0o, rws)])

        @pl.when(cid == 1)
        def _():
            pltpu.sync_copy(table.at[pl.ds(r0o, rws)], p1_h.at[pl.ds(r0o, rws)])

    return k(he_flat, dst_s2, zeros)


def kernel(x, edge_attr, edge_index, params):
    src = edge_index[0].astype(jnp.int32)
    dst = edge_index[1].astype(jnp.int32)
    pad = E_PAD - N_EDGES
    src_g = jnp.concatenate([src, jnp.zeros((pad,), jnp.int32)])
    dst_g = jnp.concatenate([dst, jnp.zeros((pad,), jnp.int32)])
    dst_s2 = jnp.concatenate(
        [dst, jnp.full((pad,), N_NODES, jnp.int32)]).reshape(E_PAD // CH, CH)
    zeros = jnp.zeros((TAB_ROWS, LAT), F32)

    en = params["enc_n"]
    ee = params["enc_e"]
    de = params["dec"]
    m128 = _bd(jnp.full((LAT, LAT), 1.0 / LAT, F32), PACK)
    m256 = _bd(jnp.full((LAT, LAT), 1.0 / LAT, F32), 8)

    hn_p = _node_encode(
        x.reshape(NPK, 4 * 128),
        _bd(en["Win"].T, PACK), _tile(en["bin"], PACK),
        _bd(en["Wout"].T, PACK), _tile(en["bout"], PACK),
        m128, _tile(en["ln_g"], PACK), _tile(en["ln_b"], PACK))

    he_half = _edge_encode(
        edge_attr.reshape(E8, 128),
        _bd(ee["Win"].T, 8), _tile(ee["bin"], 8), _bd(ee["Wout"].T, 8),
        _tile(ee["bout"], 8), m256, _tile(ee["ln_g"], 8), _tile(ee["ln_b"], 8))
    he_p = he_half.reshape(EPP, 128)

    for i in range(len(params["proc_e"])):
        pe = params["proc_e"][i]
        pn = params["proc_n"][i]

        tab = hn_p.reshape(N_NODES, LAT).astype(BF16)
        gs, gd = _sc_gather(tab, src_g, dst_g)
        he_p = _edge_mlp(
            he_p, gs.reshape(EPP, 128), gd.reshape(EPP, 128),
            _bd(pe["Win"][:, :LAT].T, PACK), _bd(pe["Win"][:, LAT:2 * LAT].T, PACK),
            _bd(pe["Win"][:, 2 * LAT:].T, PACK), _tile(pe["bin"], PACK),
            _bd(pe["Wout"].T, PACK), _tile(pe["bout"], PACK), m128,
            _tile(pe["ln_g"], PACK), _tile(pe["ln_b"], PACK))

        p0, p1 = _sc_scatter(he_p.reshape(E_PAD, LAT), dst_s2, zeros)
        hn_p = _node_update(
            hn_p, p0.reshape(NPK, 128), p1.reshape(NPK, 128),
            _bd(pn["Win"][:, :LAT].T, PACK), _bd(pn["Win"][:, LAT:].T, PACK),
            _tile(pn["bin"], PACK), _bd(pn["Wout"].T, PACK),
            _tile(pn["bout"], PACK), m128,
            _tile(pn["ln_g"], PACK), _tile(pn["ln_b"], PACK))

    out = _decode(hn_p, _bd(de["Win"].T, PACK), _tile(de["bin"], PACK),
                  _bd(de["Wout"].T, PACK), _tile(de["bout"], PACK))
    return out.reshape(N_NODES, 4)


# speedup vs baseline: 1.6838x; 1.6838x over previous
import functools

import jax
import jax.numpy as jnp
from jax import lax
from jax.experimental import pallas as pl
from jax.experimental.pallas import tpu as pltpu
from jax.experimental.pallas import tpu_sc as plsc

F32 = jnp.float32
BF16 = jnp.bfloat16
N_NODES = 10000
N_EDGES = 320000
LAT = 32
PACK = 4
NPK = N_NODES // PACK

NW = 32
E_PAD = 327680
EW = E_PAD // NW
EPP = E_PAD // PACK
EDGE_BLK = 2048
E8 = N_EDGES // 8
E8_BLK = 2000
TAB_ROWS = N_NODES + 16

G_SUP = 640
G_NSUP = EW // G_SUP
S_SUP = 1280
S_NSUP = EW // S_SUP
CH = 128


@functools.cache
def _mesh():
    return plsc.VectorSubcoreMesh(
        core_axis_name="c", subcore_axis_name="s", num_cores=2, num_subcores=16)


def _bd(w, n):
    return jax.scipy.linalg.block_diag(*([w] * n))


def _tile(v, n):
    return jnp.tile(v, n)[None, :]


def _leaky(u):
    return jnp.maximum(u, 0.01 * u)


def _ln_groups(v, m, g, b):
    mu = jnp.dot(v, m, preferred_element_type=F32)
    d = v - mu
    var = jnp.dot(d * d, m, preferred_element_type=F32)
    return d * lax.rsqrt(var + 1e-5) * g + b


def _mlp2_body(x_ref, a1_ref, b1_ref, a2_ref, b2_ref, m_ref, g_ref, be_ref, o_ref):
    u = jnp.dot(x_ref[...], a1_ref[...], preferred_element_type=F32) + b1_ref[...]
    t = _leaky(u)
    v = jnp.dot(t, a2_ref[...], preferred_element_type=F32) + b2_ref[...]
    o_ref[...] = _ln_groups(v, m_ref[...], g_ref[...], be_ref[...])


def _node_encode(x4, a1, b1, a2, b2, m, g, be):
    return pl.pallas_call(
        _mlp2_body,
        out_shape=jax.ShapeDtypeStruct((NPK, 128), F32),
    )(x4, a1, b1, a2, b2, m, g, be)


def _edge_encode(ea8, a1, b1, a2, b2, m, g, be):
    grid = E8 // E8_BLK
    full = lambda shape: pl.BlockSpec(shape, lambda i: (0, 0))
    return pl.pallas_call(
        _mlp2_body,
        grid=(grid,),
        in_specs=[
            pl.BlockSpec((E8_BLK, 128), lambda i: (i, 0)),
            full((128, 256)), full((1, 256)), full((256, 256)), full((1, 256)),
            full((256, 256)), full((1, 256)), full((1, 256)),
        ],
        out_specs=pl.BlockSpec((E8_BLK, 256), lambda i: (i, 0)),
        out_shape=jax.ShapeDtypeStruct((EPP // 2, 256), F32),
    )(ea8, a1, b1, a2, b2, m, g, be)


def _edge_mlp_body(he_ref, gs_ref, gd_ref, aa_ref, ab_ref, ac_ref, b1_ref,
                   ao_ref, b2_ref, m_ref, g_ref, be_ref, o_ref):
    he = he_ref[...]
    u = (jnp.dot(he, aa_ref[...], preferred_element_type=F32)
         + jnp.dot(gs_ref[...], ab_ref[...], preferred_element_type=F32)
         + jnp.dot(gd_ref[...], ac_ref[...], preferred_element_type=F32)
         + b1_ref[...])
    t = _leaky(u)
    v = jnp.dot(t, ao_ref[...], preferred_element_type=F32) + b2_ref[...]
    o_ref[...] = _ln_groups(v, m_ref[...], g_ref[...], be_ref[...]) + he


def _edge_mlp(he_p, gs_p, gd_p, aa, ab, ac, b1, ao, b2, m, g, be):
    grid = EPP // EDGE_BLK
    data = pl.BlockSpec((EDGE_BLK, 128), lambda i: (i, 0))
    full = lambda shape: pl.BlockSpec(shape, lambda i: (0, 0))
    return pl.pallas_call(
        _edge_mlp_body,
        grid=(grid,),
        in_specs=[data, data, data,
                  full((128, 128)), full((128, 128)), full((128, 128)),
                  full((1, 128)), full((128, 128)), full((1, 128)),
                  full((128, 128)), full((1, 128)), full((1, 128))],
        out_specs=data,
        out_shape=jax.ShapeDtypeStruct((EPP, 128), F32),
    )(he_p, gs_p, gd_p, aa, ab, ac, b1, ao, b2, m, g, be)


def _node_upd_body(hn_ref, p0_ref, p1_ref, a1_ref, a2_ref, b1_ref, ao_ref,
                   b2_ref, m_ref, g_ref, be_ref, o_ref):
    hn = hn_ref[...]
    pe = p0_ref[...] + p1_ref[...]
    u = (jnp.dot(hn, a1_ref[...], preferred_element_type=F32)
         + jnp.dot(pe, a2_ref[...], preferred_element_type=F32)
         + b1_ref[...])
    t = _leaky(u)
    v = jnp.dot(t, ao_ref[...], preferred_element_type=F32) + b2_ref[...]
    o_ref[...] = _ln_groups(v, m_ref[...], g_ref[...], be_ref[...]) + hn


def _node_update(hn_p, p0_p, p1_p, a1, a2, b1, ao, b2, m, g, be):
    return pl.pallas_call(
        _node_upd_body,
        out_shape=jax.ShapeDtypeStruct((NPK, 128), F32),
    )(hn_p, p0_p, p1_p, a1, a2, b1, ao, b2, m, g, be)


def _dec_body(hn_ref, a1_ref, b1_ref, ao_ref, b2_ref, o_ref):
    u = jnp.dot(hn_ref[...], a1_ref[...], preferred_element_type=F32) + b1_ref[...]
    t = _leaky(u)
    o_ref[...] = jnp.dot(t, ao_ref[...], preferred_element_type=F32) + b2_ref[...]


def _decode(hn_p, a1, b1, ao, b2):
    return pl.pallas_call(
        _dec_body,
        out_shape=jax.ShapeDtypeStruct((NPK, 4 * PACK), F32),
    )(hn_p, a1, b1, ao, b2)


def _sc_gather(tab, src_g, dst_g):
    @functools.partial(
        pl.kernel,
        out_type=(jax.ShapeDtypeStruct((E_PAD, LAT), F32),
                  jax.ShapeDtypeStruct((E_PAD, LAT), F32)),
        mesh=_mesh(),
        scratch_types=[
            pltpu.VMEM((EW,), jnp.int32),
            pltpu.VMEM((EW,), jnp.int32),
            pltpu.VMEM((G_SUP, LAT), F32),
            pltpu.VMEM((G_SUP, LAT), F32),
            pltpu.VMEM((G_SUP, LAT), F32),
            pltpu.VMEM((G_SUP, LAT), F32),
            pltpu.VMEM_SHARED((N_NODES, LAT), F32),
            pltpu.SemaphoreType.DMA,
            pltpu.SemaphoreType.DMA,
            pltpu.SemaphoreType.DMA,
        ],
        compiler_params=pltpu.CompilerParams(use_tc_tiling_on_sc=False),
    )
    def k(tab_h, src_h, dst_h, gs_h, gd_h, idxs, idxd,
          bs0, bs1, bd0, bd1, tab_sp, semS, semD, semW):
        cid = lax.axis_index("c")
        sid = lax.axis_index("s")
        wbase = (sid * 2 + cid) * EW

        @pl.when(sid == 0)
        def _():
            pltpu.sync_copy(tab_h, tab_sp)

        pltpu.sync_copy(src_h.at[pl.ds(wbase, EW)], idxs)
        pltpu.sync_copy(dst_h.at[pl.ds(wbase, EW)], idxd)
        plsc.subcore_barrier()
        bufs = (bs0, bs1)
        bufd = (bd0, bd1)

        def issue_gathers(s):
            ds_ = []
            for j in range(G_SUP // CH):
                o = s * G_SUP + j * CH
                ds_.append(pltpu.async_copy(
                    tab_sp.at[idxs.at[pl.ds(o, CH)]],
                    bufs[s % 2].at[pl.ds(j * CH, CH)], semS))
                ds_.append(pltpu.async_copy(
                    tab_sp.at[idxd.at[pl.ds(o, CH)]],
                    bufd[s % 2].at[pl.ds(j * CH, CH)], semD))
            return ds_

        def issue_writes(s):
            o = wbase + s * G_SUP
            return [pltpu.async_copy(bufs[s % 2], gs_h.at[pl.ds(o, G_SUP)], semW),
                    pltpu.async_copy(bufd[s % 2], gd_h.at[pl.ds(o, G_SUP)], semW)]

        g = {0: issue_gathers(0)}
        w = {}
        for s in range(G_NSUP):
            if s + 1 < G_NSUP:
                if s >= 1:
                    for d in w[s - 1]:
                        d.wait()
                g[s + 1] = issue_gathers(s + 1)
            for d in g[s]:
                d.wait()
            w[s] = issue_writes(s)
        for d in w[G_NSUP - 2] + w[G_NSUP - 1]:
            d.wait()

    return k(tab, src_g, dst_g)


def _sc_scatter(he_flat, dst_s2, zeros):
    @functools.partial(
        pl.kernel,
        out_type=(jax.ShapeDtypeStruct((N_NODES, LAT), F32),
                  jax.ShapeDtypeStruct((N_NODES, LAT), F32)),
        mesh=_mesh(),
        scratch_types=[
            pltpu.VMEM((EW // CH, CH), jnp.int32),
            pltpu.VMEM((S_SUP, LAT), F32),
            pltpu.VMEM((S_SUP, LAT), F32),
            pltpu.VMEM_SHARED((TAB_ROWS, LAT), F32),
            pltpu.SemaphoreType.DMA,
            pltpu.SemaphoreType.DMA,
        ],
        compiler_params=pltpu.CompilerParams(use_tc_tiling_on_sc=False),
    )
    def k(he_h, dst2_h, z_h, p0_h, p1_h, idx2, r0, r1, table, semL, semA):
        cid = lax.axis_index("c")
        sid = lax.axis_index("s")
        wid = sid * 2 + cid

        @pl.when(sid == 0)
        def _():
            pltpu.sync_copy(z_h, table)

        plsc.subcore_barrier()
        pltpu.sync_copy(dst2_h.at[pl.ds(wid * (EW // CH), EW // CH)], idx2)
        rows = (r0, r1)

        def issue_load(s):
            return pltpu.async_copy(
                he_h.at[pl.ds(wid * EW + s * S_SUP, S_SUP)], rows[s % 2], semL)

        def issue_adds(s):
            ds_ = []
            for j in range(S_SUP // CH):
                ds_.append(pltpu.async_copy(
                    rows[s % 2].at[pl.ds(j * CH, CH)],
                    table.at[idx2.at[s * (S_SUP // CH) + j]], semA, add=True))
            return ds_

        ld = {0: issue_load(0)}
        ad = {}
        for s in range(S_NSUP):
            ld[s].wait()
            if s + 1 < S_NSUP:
                if s >= 1:
                    for d in ad[s - 1]:
                        d.wait()
                ld[s + 1] = issue_load(s + 1)
            ad[s] = issue_adds(s)
        for d in ad[S_NSUP - 2] + ad[S_NSUP - 1]:
            d.wait()
        plsc.subcore_barrier()

        rws = N_NODES // 16
        r0o = sid * rws

        @pl.when(cid == 0)
        def _():
            pltpu.sync_copy(table.at[pl.ds(r0o, rws)], p0_h.at[pl.ds(r0o, rws)])

        @pl.when(cid == 1)
        def _():
            pltpu.sync_copy(table.at[pl.ds(r0o, rws)], p1_h.at[pl.ds(r0o, rws)])

    return k(he_flat, dst_s2, zeros)


def kernel(x, edge_attr, edge_index, params):
    src = edge_index[0].astype(jnp.int32)
    dst = edge_index[1].astype(jnp.int32)
    pad = E_PAD - N_EDGES
    src_g = jnp.concatenate([src, jnp.zeros((pad,), jnp.int32)])
    dst_g = jnp.concatenate([dst, jnp.zeros((pad,), jnp.int32)])
    dst_s2 = jnp.concatenate(
        [dst, jnp.full((pad,), N_NODES, jnp.int32)]).reshape(E_PAD // CH, CH)
    zeros = jnp.zeros((TAB_ROWS, LAT), F32)

    en = params["enc_n"]
    ee = params["enc_e"]
    de = params["dec"]
    m128 = _bd(jnp.full((LAT, LAT), 1.0 / LAT, F32), PACK)
    m256 = _bd(jnp.full((LAT, LAT), 1.0 / LAT, F32), 8)

    hn_p = _node_encode(
        x.reshape(NPK, 4 * 128),
        _bd(en["Win"].T, PACK), _tile(en["bin"], PACK),
        _bd(en["Wout"].T, PACK), _tile(en["bout"], PACK),
        m128, _tile(en["ln_g"], PACK), _tile(en["ln_b"], PACK))

    he_half = _edge_encode(
        edge_attr.reshape(E8, 128),
        _bd(ee["Win"].T, 8), _tile(ee["bin"], 8), _bd(ee["Wout"].T, 8),
        _tile(ee["bout"], 8), m256, _tile(ee["ln_g"], 8), _tile(ee["ln_b"], 8))
    he_p = he_half.reshape(EPP, 128)

    for i in range(len(params["proc_e"])):
        pe = params["proc_e"][i]
        pn = params["proc_n"][i]

        tab = hn_p.reshape(N_NODES, LAT)
        gs, gd = _sc_gather(tab, src_g, dst_g)
        he_p = _edge_mlp(
            he_p, gs.reshape(EPP, 128), gd.reshape(EPP, 128),
            _bd(pe["Win"][:, :LAT].T, PACK), _bd(pe["Win"][:, LAT:2 * LAT].T, PACK),
            _bd(pe["Win"][:, 2 * LAT:].T, PACK), _tile(pe["bin"], PACK),
            _bd(pe["Wout"].T, PACK), _tile(pe["bout"], PACK), m128,
            _tile(pe["ln_g"], PACK), _tile(pe["ln_b"], PACK))

        p0, p1 = _sc_scatter(he_p.reshape(E_PAD, LAT), dst_s2, zeros)
        hn_p = _node_update(
            hn_p, p0.reshape(NPK, 128), p1.reshape(NPK, 128),
            _bd(pn["Win"][:, :LAT].T, PACK), _bd(pn["Win"][:, LAT:].T, PACK),
            _tile(pn["bin"], PACK), _bd(pn["Wout"].T, PACK),
            _tile(pn["bout"], PACK), m128,
            _tile(pn["ln_g"], PACK), _tile(pn["ln_b"], PACK))

    out = _decode(hn_p, _bd(de["Win"].T, PACK), _tile(de["bin"], PACK),
                  _bd(de["Wout"].T, PACK), _tile(de["bout"], PACK))
    return out.reshape(N_NODES, 4)


# speedup vs baseline: 1.8291x; 1.0863x over previous
import functools

import jax
import jax.numpy as jnp
from jax import lax
from jax.experimental import pallas as pl
from jax.experimental.pallas import tpu as pltpu
from jax.experimental.pallas import tpu_sc as plsc

F32 = jnp.float32
BF16 = jnp.bfloat16
N_NODES = 10000
N_EDGES = 320000
LAT = 32
PACK = 4
NPK = N_NODES // PACK

NW = 32
E_PAD = 327680
EW = E_PAD // NW
EPP = E_PAD // PACK
EDGE_BLK = 2048
E8 = N_EDGES // 8
E8_BLK = 2000
TAB_ROWS = N_NODES + 16

G_SUP = 640
G_NSUP = EW // G_SUP
S_SUP = 1280
S_NSUP = EW // S_SUP
CH = 128
E_HALF = E_PAD // 2
EW_H = EW // 2
G_NSUP_H = EW_H // G_SUP
S_NSUP_H = EW_H // S_SUP
OFF_B = (E_HALF // PACK) // EDGE_BLK


@functools.cache
def _mesh():
    return plsc.VectorSubcoreMesh(
        core_axis_name="c", subcore_axis_name="s", num_cores=2, num_subcores=16)


def _bd(w, n):
    return jax.scipy.linalg.block_diag(*([w] * n))


def _tile(v, n):
    return jnp.tile(v, n)[None, :]


def _leaky(u):
    return jnp.maximum(u, 0.01 * u)


def _ln_groups(v, m, g, b):
    mu = jnp.dot(v, m, preferred_element_type=F32)
    d = v - mu
    var = jnp.dot(d * d, m, preferred_element_type=F32)
    return d * lax.rsqrt(var + 1e-5) * g + b


def _mlp2_body(x_ref, a1_ref, b1_ref, a2_ref, b2_ref, m_ref, g_ref, be_ref, o_ref):
    u = jnp.dot(x_ref[...], a1_ref[...], preferred_element_type=F32) + b1_ref[...]
    t = _leaky(u)
    v = jnp.dot(t, a2_ref[...], preferred_element_type=F32) + b2_ref[...]
    o_ref[...] = _ln_groups(v, m_ref[...], g_ref[...], be_ref[...])


def _node_encode(x4, a1, b1, a2, b2, m, g, be):
    return pl.pallas_call(
        _mlp2_body,
        out_shape=jax.ShapeDtypeStruct((NPK, 128), F32),
    )(x4, a1, b1, a2, b2, m, g, be)


def _edge_encode(ea8, a1, b1, a2, b2, m, g, be):
    grid = E8 // E8_BLK
    full = lambda shape: pl.BlockSpec(shape, lambda i: (0, 0))
    return pl.pallas_call(
        _mlp2_body,
        grid=(grid,),
        in_specs=[
            pl.BlockSpec((E8_BLK, 128), lambda i: (i, 0)),
            full((128, 256)), full((1, 256)), full((256, 256)), full((1, 256)),
            full((256, 256)), full((1, 256)), full((1, 256)),
        ],
        out_specs=pl.BlockSpec((E8_BLK, 256), lambda i: (i, 0)),
        out_shape=jax.ShapeDtypeStruct((EPP // 2, 256), F32),
    )(ea8, a1, b1, a2, b2, m, g, be)


def _edge_mlp_body(he_ref, gs_ref, gd_ref, aa_ref, ab_ref, ac_ref, b1_ref,
                   ao_ref, b2_ref, m_ref, g_ref, be_ref, o_ref):
    he = he_ref[...]
    u = (jnp.dot(he, aa_ref[...], preferred_element_type=F32)
         + jnp.dot(gs_ref[...], ab_ref[...], preferred_element_type=F32)
         + jnp.dot(gd_ref[...], ac_ref[...], preferred_element_type=F32)
         + b1_ref[...])
    t = _leaky(u)
    v = jnp.dot(t, ao_ref[...], preferred_element_type=F32) + b2_ref[...]
    o_ref[...] = _ln_groups(v, m_ref[...], g_ref[...], be_ref[...]) + he


def _edge_mlp(he_p, gs_p, gd_p, aa, ab, ac, b1, ao, b2, m, g, be, h):
    o = h * OFF_B
    data = pl.BlockSpec((EDGE_BLK, 128), lambda i: (i + o, 0))
    full = lambda shape: pl.BlockSpec(shape, lambda i: (0, 0))
    return pl.pallas_call(
        _edge_mlp_body,
        grid=(OFF_B,),
        in_specs=[data, data, data,
                  full((128, 128)), full((128, 128)), full((128, 128)),
                  full((1, 128)), full((128, 128)), full((1, 128)),
                  full((128, 128)), full((1, 128)), full((1, 128))],
        out_specs=data,
        out_shape=jax.ShapeDtypeStruct((EPP, 128), F32),
    )(he_p, gs_p, gd_p, aa, ab, ac, b1, ao, b2, m, g, be)


def _node_upd_body(hn_ref, p0_ref, p1_ref, p2_ref, p3_ref, a1_ref, a2_ref,
                   b1_ref, ao_ref, b2_ref, m_ref, g_ref, be_ref, o_ref):
    hn = hn_ref[...]
    pe = (p0_ref[...] + p1_ref[...]) + (p2_ref[...] + p3_ref[...])
    u = (jnp.dot(hn, a1_ref[...], preferred_element_type=F32)
         + jnp.dot(pe, a2_ref[...], preferred_element_type=F32)
         + b1_ref[...])
    t = _leaky(u)
    v = jnp.dot(t, ao_ref[...], preferred_element_type=F32) + b2_ref[...]
    o_ref[...] = _ln_groups(v, m_ref[...], g_ref[...], be_ref[...]) + hn


def _node_update(hn_p, p0_p, p1_p, p2_p, p3_p, a1, a2, b1, ao, b2, m, g, be):
    return pl.pallas_call(
        _node_upd_body,
        out_shape=jax.ShapeDtypeStruct((NPK, 128), F32),
    )(hn_p, p0_p, p1_p, p2_p, p3_p, a1, a2, b1, ao, b2, m, g, be)


def _dec_body(hn_ref, a1_ref, b1_ref, ao_ref, b2_ref, o_ref):
    u = jnp.dot(hn_ref[...], a1_ref[...], preferred_element_type=F32) + b1_ref[...]
    t = _leaky(u)
    o_ref[...] = jnp.dot(t, ao_ref[...], preferred_element_type=F32) + b2_ref[...]


def _decode(hn_p, a1, b1, ao, b2):
    return pl.pallas_call(
        _dec_body,
        out_shape=jax.ShapeDtypeStruct((NPK, 4 * PACK), F32),
    )(hn_p, a1, b1, ao, b2)


def _sc_gather(tab, src_g, dst_g, h):
    @functools.partial(
        pl.kernel,
        out_type=(jax.ShapeDtypeStruct((E_PAD, LAT), F32),
                  jax.ShapeDtypeStruct((E_PAD, LAT), F32)),
        mesh=_mesh(),
        scratch_types=[
            pltpu.VMEM((EW_H,), jnp.int32),
            pltpu.VMEM((EW_H,), jnp.int32),
            pltpu.VMEM((G_SUP, LAT), F32),
            pltpu.VMEM((G_SUP, LAT), F32),
            pltpu.VMEM((G_SUP, LAT), F32),
            pltpu.VMEM((G_SUP, LAT), F32),
            pltpu.VMEM_SHARED((N_NODES, LAT), F32),
            pltpu.SemaphoreType.DMA,
            pltpu.SemaphoreType.DMA,
            pltpu.SemaphoreType.DMA,
        ],
        compiler_params=pltpu.CompilerParams(use_tc_tiling_on_sc=False),
    )
    def k(tab_h, src_h, dst_h, gs_h, gd_h, idxs, idxd,
          bs0, bs1, bd0, bd1, tab_sp, semS, semD, semW):
        cid = lax.axis_index("c")
        sid = lax.axis_index("s")
        wbase = h * E_HALF + (sid * 2 + cid) * EW_H

        @pl.when(sid == 0)
        def _():
            pltpu.sync_copy(tab_h, tab_sp)

        pltpu.sync_copy(src_h.at[pl.ds(wbase, EW_H)], idxs)
        pltpu.sync_copy(dst_h.at[pl.ds(wbase, EW_H)], idxd)
        plsc.subcore_barrier()
        bufs = (bs0, bs1)
        bufd = (bd0, bd1)

        def issue_gathers(s):
            ds_ = []
            for j in range(G_SUP // CH):
                o = s * G_SUP + j * CH
                ds_.append(pltpu.async_copy(
                    tab_sp.at[idxs.at[pl.ds(o, CH)]],
                    bufs[s % 2].at[pl.ds(j * CH, CH)], semS))
                ds_.append(pltpu.async_copy(
                    tab_sp.at[idxd.at[pl.ds(o, CH)]],
                    bufd[s % 2].at[pl.ds(j * CH, CH)], semD))
            return ds_

        def issue_writes(s):
            o = wbase + s * G_SUP
            return [pltpu.async_copy(bufs[s % 2], gs_h.at[pl.ds(o, G_SUP)], semW),
                    pltpu.async_copy(bufd[s % 2], gd_h.at[pl.ds(o, G_SUP)], semW)]

        g = {0: issue_gathers(0)}
        w = {}
        for s in range(G_NSUP_H):
            if s + 1 < G_NSUP_H:
                if s >= 1:
                    for d in w[s - 1]:
                        d.wait()
                g[s + 1] = issue_gathers(s + 1)
            for d in g[s]:
                d.wait()
            w[s] = issue_writes(s)
        for d in w[G_NSUP_H - 2] + w[G_NSUP_H - 1]:
            d.wait()

    return k(tab, src_g, dst_g)


def _sc_scatter(he_flat, dst_s2, zeros, h):
    @functools.partial(
        pl.kernel,
        out_type=(jax.ShapeDtypeStruct((N_NODES, LAT), F32),
                  jax.ShapeDtypeStruct((N_NODES, LAT), F32)),
        mesh=_mesh(),
        scratch_types=[
            pltpu.VMEM((EW_H // CH, CH), jnp.int32),
            pltpu.VMEM((S_SUP, LAT), F32),
            pltpu.VMEM((S_SUP, LAT), F32),
            pltpu.VMEM_SHARED((TAB_ROWS, LAT), F32),
            pltpu.SemaphoreType.DMA,
            pltpu.SemaphoreType.DMA,
        ],
        compiler_params=pltpu.CompilerParams(use_tc_tiling_on_sc=False),
    )
    def k(he_h, dst2_h, z_h, p0_h, p1_h, idx2, r0, r1, table, semL, semA):
        cid = lax.axis_index("c")
        sid = lax.axis_index("s")
        wid = sid * 2 + cid

        @pl.when(sid == 0)
        def _():
            pltpu.sync_copy(z_h, table)

        plsc.subcore_barrier()
        pltpu.sync_copy(
            dst2_h.at[pl.ds(h * (E_HALF // CH) + wid * (EW_H // CH), EW_H // CH)],
            idx2)
        rows = (r0, r1)

        def issue_load(s):
            return pltpu.async_copy(
                he_h.at[pl.ds(h * E_HALF + wid * EW_H + s * S_SUP, S_SUP)],
                rows[s % 2], semL)

        def issue_adds(s):
            ds_ = []
            for j in range(S_SUP // CH):
                ds_.append(pltpu.async_copy(
                    rows[s % 2].at[pl.ds(j * CH, CH)],
                    table.at[idx2.at[s * (S_SUP // CH) + j]], semA, add=True))
            return ds_

        ld = {0: issue_load(0)}
        ad = {}
        for s in range(S_NSUP_H):
            ld[s].wait()
            if s + 1 < S_NSUP_H:
                if s >= 1:
                    for d in ad[s - 1]:
                        d.wait()
                ld[s + 1] = issue_load(s + 1)
            ad[s] = issue_adds(s)
        for d in ad[S_NSUP_H - 2] + ad[S_NSUP_H - 1]:
            d.wait()
        plsc.subcore_barrier()

        rws = N_NODES // 16
        r0o = sid * rws

        @pl.when(cid == 0)
        def _():
            pltpu.sync_copy(table.at[pl.ds(r0o, rws)], p0_h.at[pl.ds(r0o, rws)])

        @pl.when(cid == 1)
        def _():
            pltpu.sync_copy(table.at[pl.ds(r0o, rws)], p1_h.at[pl.ds(r0o, rws)])

    return k(he_flat, dst_s2, zeros)


def kernel(x, edge_attr, edge_index, params):
    src = edge_index[0].astype(jnp.int32)
    dst = edge_index[1].astype(jnp.int32)
    pad = E_PAD - N_EDGES
    src_g = jnp.concatenate([src, jnp.zeros((pad,), jnp.int32)])
    dst_g = jnp.concatenate([dst, jnp.zeros((pad,), jnp.int32)])
    dst_s2 = jnp.concatenate(
        [dst, jnp.full((pad,), N_NODES, jnp.int32)]).reshape(E_PAD // CH, CH)
    zeros = jnp.zeros((TAB_ROWS, LAT), F32)

    en = params["enc_n"]
    ee = params["enc_e"]
    de = params["dec"]
    m128 = _bd(jnp.full((LAT, LAT), 1.0 / LAT, F32), PACK)
    m256 = _bd(jnp.full((LAT, LAT), 1.0 / LAT, F32), 8)

    hn_p = _node_encode(
        x.reshape(NPK, 4 * 128),
        _bd(en["Win"].T, PACK), _tile(en["bin"], PACK),
        _bd(en["Wout"].T, PACK), _tile(en["bout"], PACK),
        m128, _tile(en["ln_g"], PACK), _tile(en["ln_b"], PACK))

    he_half = _edge_encode(
        edge_attr.reshape(E8, 128),
        _bd(ee["Win"].T, 8), _tile(ee["bin"], 8), _bd(ee["Wout"].T, 8),
        _tile(ee["bout"], 8), m256, _tile(ee["ln_g"], 8), _tile(ee["ln_b"], 8))
    he_p = he_half.reshape(EPP, 128)

    he_a = he_b = he_p
    for i in range(len(params["proc_e"])):
        pe = params["proc_e"][i]
        pn = params["proc_n"][i]
        ew = (_bd(pe["Win"][:, :LAT].T, PACK),
              _bd(pe["Win"][:, LAT:2 * LAT].T, PACK),
              _bd(pe["Win"][:, 2 * LAT:].T, PACK), _tile(pe["bin"], PACK),
              _bd(pe["Wout"].T, PACK), _tile(pe["bout"], PACK), m128,
              _tile(pe["ln_g"], PACK), _tile(pe["ln_b"], PACK))

        tab = hn_p.reshape(N_NODES, LAT)
        gsa, gda = _sc_gather(tab, src_g, dst_g, 0)
        gsb, gdb = _sc_gather(tab, src_g, dst_g, 1)
        he_a = _edge_mlp(
            he_a, gsa.reshape(EPP, 128), gda.reshape(EPP, 128), *ew, h=0)
        pa0, pa1 = _sc_scatter(he_a.reshape(E_PAD, LAT), dst_s2, zeros, 0)
        he_b = _edge_mlp(
            he_b, gsb.reshape(EPP, 128), gdb.reshape(EPP, 128), *ew, h=1)
        pb0, pb1 = _sc_scatter(he_b.reshape(E_PAD, LAT), dst_s2, zeros, 1)
        hn_p = _node_update(
            hn_p, pa0.reshape(NPK, 128), pa1.reshape(NPK, 128),
            pb0.reshape(NPK, 128), pb1.reshape(NPK, 128),
            _bd(pn["Win"][:, :LAT].T, PACK), _bd(pn["Win"][:, LAT:].T, PACK),
            _tile(pn["bin"], PACK), _bd(pn["Wout"].T, PACK),
            _tile(pn["bout"], PACK), m128,
            _tile(pn["ln_g"], PACK), _tile(pn["ln_b"], PACK))

    out = _decode(hn_p, _bd(de["Win"].T, PACK), _tile(de["bin"], PACK),
                  _bd(de["Wout"].T, PACK), _tile(de["bout"], PACK))
    return out.reshape(N_NODES, 4)
